# initial kernel scaffold (unmeasured)
import functools

import jax
import jax.numpy as jnp
from jax import lax
from jax.experimental import pallas as pl
from jax.experimental.pallas import tpu as pltpu

N_DEV = 4
N_LOCAL_E = 8
N_EXPERTS = N_DEV * N_LOCAL_E
CAP = 128


def _moe_body(disp_ref, x_ref, sw_ref, w_ref,
              yout_ref, shared_ref,
              recv_ref, send_sem, recv_sem, send_sem2, recv_sem2):
    me = lax.axis_index("i")

    bsem = pltpu.get_barrier_semaphore()
    for j in range(1, N_DEV):
        pl.semaphore_signal(
            bsem, inc=1,
            device_id=((me + j) % N_DEV,),
            device_id_type=pl.DeviceIdType.MESH,
        )
    pl.semaphore_wait(bsem, N_DEV - 1)

    sends = []
    for j in range(1, N_DEV):
        r = pltpu.make_async_remote_copy(
            src_ref=disp_ref.at[j],
            dst_ref=recv_ref.at[N_DEV - j],
            send_sem=send_sem.at[j],
            recv_sem=recv_sem.at[N_DEV - j],
            device_id=((me + j) % N_DEV,),
            device_id_type=pl.DeviceIdType.MESH,
        )
        r.start()
        sends.append(r)

    shared_ref[...] = jnp.dot(
        x_ref[...], sw_ref[...], preferred_element_type=jnp.float32
    )
    for k in range(N_LOCAL_E):
        yout_ref[0, k] = jnp.dot(
            disp_ref[0, k], w_ref[k], preferred_element_type=jnp.float32
        ).astype(jnp.bfloat16)

    sends2 = []
    for j in range(1, N_DEV):
        recv_only = pltpu.make_async_remote_copy(
            src_ref=disp_ref.at[j],
            dst_ref=recv_ref.at[j],
            send_sem=send_sem.at[j],
            recv_sem=recv_sem.at[j],
            device_id=((me + j) % N_DEV,),
            device_id_type=pl.DeviceIdType.MESH,
        )
        recv_only.wait_recv()
        for k in range(N_LOCAL_E):
            recv_ref[j, k] = jnp.dot(
                recv_ref[j, k], w_ref[k], preferred_element_type=jnp.float32
            ).astype(jnp.bfloat16)
        r2 = pltpu.make_async_remote_copy(
            src_ref=recv_ref.at[j],
            dst_ref=yout_ref.at[N_DEV - j],
            send_sem=send_sem2.at[j],
            recv_sem=recv_sem2.at[N_DEV - j],
            device_id=((me + j) % N_DEV,),
            device_id_type=pl.DeviceIdType.MESH,
        )
        r2.start()
        sends2.append(r2)

    for r in sends:
        r.wait_send()
    for r in sends2:
        r.wait_send()
    for j in range(1, N_DEV):
        recv2_only = pltpu.make_async_remote_copy(
            src_ref=recv_ref.at[j],
            dst_ref=yout_ref.at[j],
            send_sem=send_sem2.at[j],
            recv_sem=recv_sem2.at[j],
            device_id=((me + j) % N_DEV,),
            device_id_type=pl.DeviceIdType.MESH,
        )
        recv2_only.wait_recv()


def kernel(x, router_W, route_idx, expert_W, shared_W):
    n_tok, d = x.shape
    me = lax.axis_index("i")

    e = route_idx[:, 0]
    scores = jnp.dot(x, router_W, preferred_element_type=jnp.float32)
    probs = jax.nn.softmax(scores, axis=-1)
    p = jnp.take_along_axis(probs, e[:, None], axis=-1)[:, 0]

    xs = (x * p[:, None]).astype(jnp.bfloat16)
    order = jnp.argsort(e)
    se = e[order]
    start = jnp.searchsorted(se, jnp.arange(N_EXPERTS, dtype=se.dtype))
    pos = jnp.arange(n_tok) - start[se]
    disp = jnp.zeros((N_EXPERTS, CAP, d), jnp.bfloat16)
    disp = disp.at[se, pos].set(xs[order], mode="drop")
    disp = disp.reshape(N_DEV, N_LOCAL_E, CAP, d)
    disp = jnp.roll(disp, -me, axis=0)

    x_bf = x.astype(jnp.bfloat16)
    sw_bf = shared_W.astype(jnp.bfloat16)
    w_bf = expert_W.astype(jnp.bfloat16)

    y_recv, shared_out = pl.pallas_call(
        _moe_body,
        out_shape=(
            jax.ShapeDtypeStruct((N_DEV, N_LOCAL_E, CAP, d), jnp.bfloat16),
            jax.ShapeDtypeStruct((n_tok, d), jnp.float32),
        ),
        in_specs=[
            pl.BlockSpec(memory_space=pltpu.VMEM),
            pl.BlockSpec(memory_space=pltpu.VMEM),
            pl.BlockSpec(memory_space=pltpu.VMEM),
            pl.BlockSpec(memory_space=pltpu.VMEM),
        ],
        out_specs=(
            pl.BlockSpec(memory_space=pltpu.VMEM),
            pl.BlockSpec(memory_space=pltpu.VMEM),
        ),
        scratch_shapes=[
            pltpu.VMEM((N_DEV, N_LOCAL_E, CAP, d), jnp.bfloat16),
            pltpu.SemaphoreType.DMA((N_DEV,)),
            pltpu.SemaphoreType.DMA((N_DEV,)),
            pltpu.SemaphoreType.DMA((N_DEV,)),
            pltpu.SemaphoreType.DMA((N_DEV,)),
        ],
        compiler_params=pltpu.CompilerParams(collective_id=0),
    )(disp, x_bf, sw_bf, w_bf)

    y_by_d = jnp.roll(y_recv, me, axis=0).reshape(N_EXPERTS, CAP, d)
    tok_y = y_by_d[se, pos].astype(jnp.float32)
    tok_y = jnp.where((pos < CAP)[:, None], tok_y, 0.0)
    out = shared_out.at[order].add(tok_y)
    return out


# baseline (device time: 295410 ns/iter reference)
import functools

import jax
import jax.numpy as jnp
from jax import lax
from jax.experimental import pallas as pl
from jax.experimental.pallas import tpu as pltpu

N_DEV = 4
N_LOCAL_E = 8
N_EXPERTS = N_DEV * N_LOCAL_E
CAP = 128


def _moe_body(disp_ref, x_ref, sw_ref, w_ref,
              yout_ref, shared_ref,
              recv_ref, send_sem, recv_sem, send_sem2, recv_sem2):
    me = lax.axis_index("i")

    bsem = pltpu.get_barrier_semaphore()
    for j in range(1, N_DEV):
        pl.semaphore_signal(
            bsem, inc=1,
            device_id=((me + j) % N_DEV,),
            device_id_type=pl.DeviceIdType.MESH,
        )
    pl.semaphore_wait(bsem, N_DEV - 1)

    sends = []
    for j in range(1, N_DEV):
        r = pltpu.make_async_remote_copy(
            src_ref=disp_ref.at[j],
            dst_ref=recv_ref.at[N_DEV - j],
            send_sem=send_sem.at[j],
            recv_sem=recv_sem.at[N_DEV - j],
            device_id=((me + j) % N_DEV,),
            device_id_type=pl.DeviceIdType.MESH,
        )
        r.start()
        sends.append(r)

    shared_ref[...] = jnp.dot(
        x_ref[...], sw_ref[...], preferred_element_type=jnp.float32
    )
    for k in range(N_LOCAL_E):
        yout_ref[0, k] = jnp.dot(
            disp_ref[0, k], w_ref[k], preferred_element_type=jnp.float32
        ).astype(jnp.bfloat16)

    sends2 = []
    for j in range(1, N_DEV):
        recv_only = pltpu.make_async_remote_copy(
            src_ref=disp_ref.at[j],
            dst_ref=recv_ref.at[j],
            send_sem=send_sem.at[j],
            recv_sem=recv_sem.at[j],
            device_id=((me + j) % N_DEV,),
            device_id_type=pl.DeviceIdType.MESH,
        )
        recv_only.wait_recv()
        for k in range(N_LOCAL_E):
            recv_ref[j, k] = jnp.dot(
                recv_ref[j, k], w_ref[k], preferred_element_type=jnp.float32
            ).astype(jnp.bfloat16)
        r2 = pltpu.make_async_remote_copy(
            src_ref=recv_ref.at[j],
            dst_ref=yout_ref.at[N_DEV - j],
            send_sem=send_sem2.at[j],
            recv_sem=recv_sem2.at[N_DEV - j],
            device_id=((me + j) % N_DEV,),
            device_id_type=pl.DeviceIdType.MESH,
        )
        r2.start()
        sends2.append(r2)

    for r in sends:
        r.wait_send()
    for r in sends2:
        r.wait_send()
    for j in range(1, N_DEV):
        recv2_only = pltpu.make_async_remote_copy(
            src_ref=recv_ref.at[j],
            dst_ref=yout_ref.at[j],
            send_sem=send_sem2.at[j],
            recv_sem=recv_sem2.at[j],
            device_id=((me + j) % N_DEV,),
            device_id_type=pl.DeviceIdType.MESH,
        )
        recv2_only.wait_recv()


def kernel(x, router_W, route_idx, expert_W, shared_W):
    n_tok, d = x.shape
    me = lax.axis_index("i")

    e = route_idx[:, 0]
    scores = jnp.dot(x, router_W, preferred_element_type=jnp.float32)
    probs = jax.nn.softmax(scores, axis=-1)
    p = jnp.take_along_axis(probs, e[:, None], axis=-1)[:, 0]

    xs = (x * p[:, None]).astype(jnp.bfloat16)
    order = jnp.argsort(e)
    se = e[order]
    start = jnp.searchsorted(se, jnp.arange(N_EXPERTS, dtype=se.dtype))
    pos = jnp.arange(n_tok) - start[se]
    disp = jnp.zeros((N_EXPERTS, CAP, d), jnp.bfloat16)
    disp = disp.at[se, pos].set(xs[order], mode="drop")
    disp = disp.reshape(N_DEV, N_LOCAL_E, CAP, d)
    disp = jnp.roll(disp, -me, axis=0)

    x_bf = x.astype(jnp.bfloat16)
    sw_bf = shared_W.astype(jnp.bfloat16)
    w_bf = expert_W.astype(jnp.bfloat16)

    y_recv, shared_out = pl.pallas_call(
        _moe_body,
        out_shape=(
            jax.ShapeDtypeStruct((N_DEV, N_LOCAL_E, CAP, d), jnp.bfloat16),
            jax.ShapeDtypeStruct((n_tok, d), jnp.float32),
        ),
        in_specs=[
            pl.BlockSpec(memory_space=pltpu.VMEM),
            pl.BlockSpec(memory_space=pltpu.VMEM),
            pl.BlockSpec(memory_space=pltpu.VMEM),
            pl.BlockSpec(memory_space=pltpu.VMEM),
        ],
        out_specs=(
            pl.BlockSpec(memory_space=pltpu.VMEM),
            pl.BlockSpec(memory_space=pltpu.VMEM),
        ),
        scratch_shapes=[
            pltpu.VMEM((N_DEV, N_LOCAL_E, CAP, d), jnp.bfloat16),
            pltpu.SemaphoreType.DMA((N_DEV,)),
            pltpu.SemaphoreType.DMA((N_DEV,)),
            pltpu.SemaphoreType.DMA((N_DEV,)),
            pltpu.SemaphoreType.DMA((N_DEV,)),
        ],
        compiler_params=pltpu.CompilerParams(
            collective_id=0,
            vmem_limit_bytes=64 * 1024 * 1024,
        ),
    )(disp, x_bf, sw_bf, w_bf)

    y_by_d = jnp.roll(y_recv, me, axis=0).reshape(N_EXPERTS, CAP, d)
    tok_y = y_by_d[se, pos].astype(jnp.float32)
    tok_y = jnp.where((pos < CAP)[:, None], tok_y, 0.0)
    out = shared_out.at[order].add(tok_y)
    return out


# device time: 252512 ns/iter; 1.1699x vs baseline; 1.1699x over previous
import functools

import jax
import jax.numpy as jnp
from jax import lax
from jax.experimental import pallas as pl
from jax.experimental.pallas import tpu as pltpu

N_DEV = 4
N_LOCAL_E = 8
N_EXPERTS = N_DEV * N_LOCAL_E
CAP = 128


def _moe_body(disp_ref, x_ref, sw_ref, w_ref,
              yout_ref, shared_ref,
              recv_ref, send_sem, recv_sem, send_sem2, recv_sem2):
    me = lax.axis_index("i")

    bsem = pltpu.get_barrier_semaphore()
    for j in range(1, N_DEV):
        pl.semaphore_signal(
            bsem, inc=1,
            device_id=((me + j) % N_DEV,),
            device_id_type=pl.DeviceIdType.MESH,
        )
    pl.semaphore_wait(bsem, N_DEV - 1)

    sends = []
    for j in range(1, N_DEV):
        r = pltpu.make_async_remote_copy(
            src_ref=disp_ref.at[j],
            dst_ref=recv_ref.at[N_DEV - j],
            send_sem=send_sem.at[j],
            recv_sem=recv_sem.at[N_DEV - j],
            device_id=((me + j) % N_DEV,),
            device_id_type=pl.DeviceIdType.MESH,
        )
        r.start()
        sends.append(r)

    shared_ref[...] = jnp.dot(
        x_ref[...], sw_ref[...], preferred_element_type=jnp.float32
    )
    for k in range(N_LOCAL_E):
        yout_ref[0, k] = jnp.dot(
            disp_ref[0, k], w_ref[k], preferred_element_type=jnp.float32
        ).astype(jnp.bfloat16)

    sends2 = []
    for j in range(1, N_DEV):
        recv_only = pltpu.make_async_remote_copy(
            src_ref=disp_ref.at[j],
            dst_ref=recv_ref.at[j],
            send_sem=send_sem.at[j],
            recv_sem=recv_sem.at[j],
            device_id=((me + j) % N_DEV,),
            device_id_type=pl.DeviceIdType.MESH,
        )
        recv_only.wait_recv()
        for k in range(N_LOCAL_E):
            recv_ref[j, k] = jnp.dot(
                recv_ref[j, k], w_ref[k], preferred_element_type=jnp.float32
            ).astype(jnp.bfloat16)
        r2 = pltpu.make_async_remote_copy(
            src_ref=recv_ref.at[j],
            dst_ref=yout_ref.at[N_DEV - j],
            send_sem=send_sem2.at[j],
            recv_sem=recv_sem2.at[N_DEV - j],
            device_id=((me + j) % N_DEV,),
            device_id_type=pl.DeviceIdType.MESH,
        )
        r2.start()
        sends2.append(r2)

    for r in sends:
        r.wait_send()
    for r in sends2:
        r.wait_send()
    for j in range(1, N_DEV):
        recv2_only = pltpu.make_async_remote_copy(
            src_ref=recv_ref.at[j],
            dst_ref=yout_ref.at[j],
            send_sem=send_sem2.at[j],
            recv_sem=recv_sem2.at[j],
            device_id=((me + j) % N_DEV,),
            device_id_type=pl.DeviceIdType.MESH,
        )
        recv2_only.wait_recv()


def kernel(x, router_W, route_idx, expert_W, shared_W):
    n_tok, d = x.shape
    me = lax.axis_index("i")

    e = route_idx[:, 0]
    scores = jnp.dot(x, router_W, preferred_element_type=jnp.float32)
    probs = jax.nn.softmax(scores, axis=-1)
    p = jnp.take_along_axis(probs, e[:, None], axis=-1)[:, 0]

    xs = (x * p[:, None]).astype(jnp.bfloat16)
    order = jnp.argsort(e)
    se = e[order]
    start = jnp.searchsorted(se, jnp.arange(N_EXPERTS, dtype=se.dtype))
    rank_sorted = jnp.arange(n_tok) - start[se]
    rank = jnp.zeros((n_tok,), jnp.int32).at[order].set(rank_sorted)

    rot = ((e // N_LOCAL_E) - me) % N_DEV
    slot = rot * (N_LOCAL_E * CAP) + (e % N_LOCAL_E) * CAP + rank
    slot = jnp.where(rank < CAP, slot, 1 << 30)
    disp = (
        jnp.zeros((N_DEV * N_LOCAL_E * CAP, d), jnp.bfloat16)
        .at[slot].set(xs, mode="drop")
        .reshape(N_DEV, N_LOCAL_E, CAP, d)
    )

    x_bf = x.astype(jnp.bfloat16)
    sw_bf = shared_W.astype(jnp.bfloat16)
    w_bf = expert_W.astype(jnp.bfloat16)

    y_recv, shared_out = pl.pallas_call(
        _moe_body,
        out_shape=(
            jax.ShapeDtypeStruct((N_DEV, N_LOCAL_E, CAP, d), jnp.bfloat16),
            jax.ShapeDtypeStruct((n_tok, d), jnp.float32),
        ),
        in_specs=[
            pl.BlockSpec(memory_space=pltpu.VMEM),
            pl.BlockSpec(memory_space=pltpu.VMEM),
            pl.BlockSpec(memory_space=pltpu.VMEM),
            pl.BlockSpec(memory_space=pltpu.VMEM),
        ],
        out_specs=(
            pl.BlockSpec(memory_space=pltpu.VMEM),
            pl.BlockSpec(memory_space=pltpu.VMEM),
        ),
        scratch_shapes=[
            pltpu.VMEM((N_DEV, N_LOCAL_E, CAP, d), jnp.bfloat16),
            pltpu.SemaphoreType.DMA((N_DEV,)),
            pltpu.SemaphoreType.DMA((N_DEV,)),
            pltpu.SemaphoreType.DMA((N_DEV,)),
            pltpu.SemaphoreType.DMA((N_DEV,)),
        ],
        compiler_params=pltpu.CompilerParams(
            collective_id=0,
            vmem_limit_bytes=64 * 1024 * 1024,
        ),
    )(disp, x_bf, sw_bf, w_bf)

    y_flat = y_recv.reshape(N_DEV * N_LOCAL_E * CAP, d)
    valid = rank < CAP
    tok_y = y_flat[jnp.where(valid, slot, 0)].astype(jnp.float32)
    out = shared_out + jnp.where(valid[:, None], tok_y, 0.0)
    return out


# device time: 236445 ns/iter; 1.2494x vs baseline; 1.0680x over previous
import jax
import jax.numpy as jnp
from jax import lax
from jax.experimental import pallas as pl
from jax.experimental.pallas import tpu as pltpu

N_DEV = 4
N_LOCAL_E = 8
N_EXPERTS = N_DEV * N_LOCAL_E
CAP = 128
N_SLOTS = N_DEV * N_LOCAL_E * CAP

_VMEM = 64 * 1024 * 1024


def _prep_body(x_ref, rw_ref, ri_ref, ohT_ref, disp_ref, p_ref, xbf_ref):
    xb = x_ref[...].astype(jnp.bfloat16)
    xbf_ref[...] = xb
    disp_ref[...] = jnp.dot(
        ohT_ref[...], xb, preferred_element_type=jnp.float32
    ).astype(jnp.bfloat16)

    scores = jnp.dot(x_ref[...], rw_ref[...], preferred_element_type=jnp.float32)
    m = jnp.max(scores, axis=-1, keepdims=True)
    ex = jnp.exp(scores - m)
    probs = ex / jnp.sum(ex, axis=-1, keepdims=True)
    eid = lax.broadcasted_iota(jnp.int32, scores.shape, 1)
    sel = (eid == ri_ref[...]).astype(jnp.float32)
    p_ref[...] = jnp.sum(probs * sel, axis=-1, keepdims=True)


def _moe_body(disp_ref, x_ref, sw_ref, w_ref,
              yout_ref, shared_ref,
              recv_ref, send_sem, recv_sem, send_sem2, recv_sem2):
    me = lax.axis_index("i")

    bsem = pltpu.get_barrier_semaphore()
    for j in range(1, N_DEV):
        pl.semaphore_signal(
            bsem, inc=1,
            device_id=((me + j) % N_DEV,),
            device_id_type=pl.DeviceIdType.MESH,
        )
    pl.semaphore_wait(bsem, N_DEV - 1)

    sends = []
    for j in range(1, N_DEV):
        r = pltpu.make_async_remote_copy(
            src_ref=disp_ref.at[j],
            dst_ref=recv_ref.at[N_DEV - j],
            send_sem=send_sem.at[j],
            recv_sem=recv_sem.at[N_DEV - j],
            device_id=((me + j) % N_DEV,),
            device_id_type=pl.DeviceIdType.MESH,
        )
        r.start()
        sends.append(r)

    shared_ref[...] = jnp.dot(
        x_ref[...], sw_ref[...], preferred_element_type=jnp.float32
    )
    for k in range(N_LOCAL_E):
        yout_ref[0, k] = jnp.dot(
            disp_ref[0, k], w_ref[k], preferred_element_type=jnp.float32
        ).astype(jnp.bfloat16)

    sends2 = []
    for j in range(1, N_DEV):
        recv_only = pltpu.make_async_remote_copy(
            src_ref=disp_ref.at[j],
            dst_ref=recv_ref.at[j],
            send_sem=send_sem.at[j],
            recv_sem=recv_sem.at[j],
            device_id=((me + j) % N_DEV,),
            device_id_type=pl.DeviceIdType.MESH,
        )
        recv_only.wait_recv()
        for k in range(N_LOCAL_E):
            recv_ref[j, k] = jnp.dot(
                recv_ref[j, k], w_ref[k], preferred_element_type=jnp.float32
            ).astype(jnp.bfloat16)
        r2 = pltpu.make_async_remote_copy(
            src_ref=recv_ref.at[j],
            dst_ref=yout_ref.at[N_DEV - j],
            send_sem=send_sem2.at[j],
            recv_sem=recv_sem2.at[N_DEV - j],
            device_id=((me + j) % N_DEV,),
            device_id_type=pl.DeviceIdType.MESH,
        )
        r2.start()
        sends2.append(r2)

    for r in sends:
        r.wait_send()
    for r in sends2:
        r.wait_send()
    for j in range(1, N_DEV):
        recv2_only = pltpu.make_async_remote_copy(
            src_ref=recv_ref.at[j],
            dst_ref=yout_ref.at[j],
            send_sem=send_sem2.at[j],
            recv_sem=recv_sem2.at[j],
            device_id=((me + j) % N_DEV,),
            device_id_type=pl.DeviceIdType.MESH,
        )
        recv2_only.wait_recv()


def _combine_body(oh_ref, y_ref, sh_ref, p_ref, out_ref):
    t = jnp.dot(oh_ref[...], y_ref[...], preferred_element_type=jnp.float32)
    out_ref[...] = sh_ref[...] + p_ref[...] * t


def kernel(x, router_W, route_idx, expert_W, shared_W):
    n_tok, d = x.shape
    me = lax.axis_index("i")

    e = route_idx[:, 0]
    oh32 = (e[:, None] == jnp.arange(N_EXPERTS, dtype=e.dtype)[None, :])
    ohf = oh32.astype(jnp.float32)
    rank = jnp.sum(ohf * (jnp.cumsum(ohf, axis=0) - 1.0), axis=1).astype(jnp.int32)
    rot = ((e // N_LOCAL_E) - me) % N_DEV
    slot = rot * (N_LOCAL_E * CAP) + (e % N_LOCAL_E) * CAP + rank
    slot = jnp.where(rank < CAP, slot, jnp.int32(1 << 30))

    srange = jnp.arange(N_SLOTS, dtype=jnp.int32)
    onehotT = (srange[:, None] == slot[None, :]).astype(jnp.bfloat16)
    onehot = (slot[:, None] == srange[None, :]).astype(jnp.bfloat16)

    sw_bf = shared_W.astype(jnp.bfloat16)
    w_bf = expert_W.astype(jnp.bfloat16)

    disp_flat, p, x_bf = pl.pallas_call(
        _prep_body,
        out_shape=(
            jax.ShapeDtypeStruct((N_SLOTS, d), jnp.bfloat16),
            jax.ShapeDtypeStruct((n_tok, 1), jnp.float32),
            jax.ShapeDtypeStruct((n_tok, d), jnp.bfloat16),
        ),
        in_specs=[pl.BlockSpec(memory_space=pltpu.VMEM)] * 4,
        out_specs=(pl.BlockSpec(memory_space=pltpu.VMEM),) * 3,
        compiler_params=pltpu.CompilerParams(vmem_limit_bytes=_VMEM),
    )(x, router_W, route_idx, onehotT)
    disp = disp_flat.reshape(N_DEV, N_LOCAL_E, CAP, d)

    y_recv, shared_out = pl.pallas_call(
        _moe_body,
        out_shape=(
            jax.ShapeDtypeStruct((N_DEV, N_LOCAL_E, CAP, d), jnp.bfloat16),
            jax.ShapeDtypeStruct((n_tok, d), jnp.float32),
        ),
        in_specs=[pl.BlockSpec(memory_space=pltpu.VMEM)] * 4,
        out_specs=(pl.BlockSpec(memory_space=pltpu.VMEM),) * 2,
        scratch_shapes=[
            pltpu.VMEM((N_DEV, N_LOCAL_E, CAP, d), jnp.bfloat16),
            pltpu.SemaphoreType.DMA((N_DEV,)),
            pltpu.SemaphoreType.DMA((N_DEV,)),
            pltpu.SemaphoreType.DMA((N_DEV,)),
            pltpu.SemaphoreType.DMA((N_DEV,)),
        ],
        compiler_params=pltpu.CompilerParams(
            collective_id=0,
            vmem_limit_bytes=_VMEM,
        ),
    )(disp, x_bf, sw_bf, w_bf)

    out = pl.pallas_call(
        _combine_body,
        out_shape=jax.ShapeDtypeStruct((n_tok, d), jnp.float32),
        in_specs=[pl.BlockSpec(memory_space=pltpu.VMEM)] * 4,
        out_specs=pl.BlockSpec(memory_space=pltpu.VMEM),
        compiler_params=pltpu.CompilerParams(vmem_limit_bytes=_VMEM),
    )(onehot, y_recv.reshape(N_SLOTS, d), shared_out, p)
    return out
